# baseline (speedup 1.0000x reference)
import functools

import jax
import jax.numpy as jnp
from jax import lax
from jax.experimental import pallas as pl
from jax.experimental.pallas import tpu as pltpu
from jax.experimental.pallas import tpu_sc as plsc

B = 16
L = 2048
D = 256

NC = 2
NS = 16
SEG = 2 * L + 1
HALF = L // 2
NPAIR = B * SEG // 2


def _sc_interleave(item3, action3, ctx3):
    mesh = plsc.VectorSubcoreMesh(core_axis_name="c", subcore_axis_name="s")

    @functools.partial(
        pl.kernel,
        mesh=mesh,
        out_type=jax.ShapeDtypeStruct((NPAIR, 2, D), jnp.float32),
        scratch_types=[
            pltpu.SemaphoreType.DMA,
            pltpu.SemaphoreType.DMA,
        ],
    )
    def k(item_hbm, action_hbm, ctx_hbm, out_hbm, sem_i, sem_a):
        wid = lax.axis_index("s") * NC + lax.axis_index("c")
        b = wid // 2
        h = wid % 2
        par = b % 2
        p0 = (b * SEG) // 2
        src = b * L + h * HALF
        p_i = p0 + par + h * HALF
        p_a = p0 + 1 + h * HALF
        slot_i = 1 - par
        slot_a = par
        cp_i = pltpu.async_copy(
            item_hbm.at[pl.ds(src, HALF)],
            out_hbm.at[pl.ds(p_i, HALF), pl.ds(slot_i, 1)], sem_i)
        cp_a = pltpu.async_copy(
            action_hbm.at[pl.ds(src, HALF)],
            out_hbm.at[pl.ds(p_a, HALF), pl.ds(slot_a, 1)], sem_a)
        cp_i.wait()
        cp_a.wait()

        @pl.when(wid < B)
        def _():
            cp0 = (wid * SEG) // 2
            pltpu.sync_copy(
                ctx_hbm.at[pl.ds(wid, 1)],
                out_hbm.at[pl.ds(cp0, 1), pl.ds(wid % 2, 1)])

    return k(item3, action3, ctx3)


def kernel(item_values, action_values, contextual_values):
    out3 = _sc_interleave(
        item_values.reshape(B * L, 1, D),
        action_values.reshape(B * L, 1, D),
        contextual_values.reshape(B, 1, D),
    )
    out_values = out3.reshape(B * SEG, D)
    out_lengths = jnp.full((B,), SEG, dtype=jnp.int32)
    out_offsets = (jnp.arange(B + 1, dtype=jnp.int32) * SEG).astype(jnp.int32)
    return out_values, out_lengths, out_offsets


# speedup vs baseline: 11.1004x; 11.1004x over previous
import functools

import jax
import jax.numpy as jnp
from jax import lax
from jax.experimental import pallas as pl
from jax.experimental.pallas import tpu as pltpu
from jax.experimental.pallas import tpu_sc as plsc

B = 16
L = 2048
D = 256

NC = 2
NS = 16
SEG = 2 * L + 1
HALF = L // 2
NPAIR = B * SEG // 2
CHUNK = 128
NCHUNK = HALF // CHUNK


def _sc_interleave(item_values, action_values, contextual_values):
    mesh = plsc.VectorSubcoreMesh(core_axis_name="c", subcore_axis_name="s")
    nbuf = 3
    nt = 2 * NCHUNK

    @functools.partial(
        pl.kernel,
        mesh=mesh,
        compiler_params=pltpu.CompilerParams(use_tc_tiling_on_sc=False),
        out_type=jax.ShapeDtypeStruct((NPAIR, 2 * D), jnp.float32),
        scratch_types=(
            [pltpu.VMEM((CHUNK, D), jnp.float32) for _ in range(nbuf)]
            + [pltpu.VMEM((1, D), jnp.float32)]
            + [pltpu.SemaphoreType.DMA for _ in range(2 * nbuf)]
        ),
    )
    def k(item_hbm, action_hbm, ctx_hbm, out_hbm, *scr):
        bufs = scr[0:nbuf]
        ctx_buf = scr[nbuf]
        gsem = scr[nbuf + 1:nbuf + 1 + nbuf]
        ssem = scr[nbuf + 1 + nbuf:]
        wid = lax.axis_index("s") * NC + lax.axis_index("c")
        b = wid // 2
        h = wid % 2
        par = b % 2
        p0 = (b * SEG) // 2
        src_base = b * L + h * HALF
        k0 = h * HALF
        p_item = p0 + par + k0
        p_act = p0 + 1 + k0
        c_item = D * (1 - par)
        c_act = D * par

        def start_gather(t):
            ref = item_hbm if t % 2 == 0 else action_hbm
            src = src_base + (t // 2) * CHUNK
            return pltpu.async_copy(ref.at[pl.ds(src, CHUNK)],
                                    bufs[t % nbuf], gsem[t % nbuf])

        def start_scatter(t):
            pb = (p_item if t % 2 == 0 else p_act) + (t // 2) * CHUNK
            col = c_item if t % 2 == 0 else c_act
            return pltpu.async_copy(
                bufs[t % nbuf],
                out_hbm.at[pl.ds(pb, CHUNK), pl.ds(col, D)],
                ssem[t % nbuf])

        gath = {t: start_gather(t) for t in range(min(nbuf, nt))}
        scat = {}
        for t in range(nt):
            if t >= 1 and t + 2 < nt:
                scat[t - 1].wait()
                gath[t + 2] = start_gather(t + 2)
            gath[t].wait()
            scat[t] = start_scatter(t)
        for t in range(max(0, nt - nbuf), nt):
            scat[t].wait()

        @pl.when(wid < B)
        def _():
            cp0 = (wid * SEG) // 2
            pltpu.sync_copy(ctx_hbm.at[pl.ds(wid, 1)], ctx_buf)
            pltpu.sync_copy(
                ctx_buf,
                out_hbm.at[pl.ds(cp0, 1), pl.ds(D * (wid % 2), D)])

    return k(item_values, action_values, contextual_values)


def kernel(item_values, action_values, contextual_values):
    out2 = _sc_interleave(item_values, action_values, contextual_values)
    out_values = out2.reshape(B * SEG, D)
    out_lengths = jnp.full((B,), SEG, dtype=jnp.int32)
    out_offsets = (jnp.arange(B + 1, dtype=jnp.int32) * SEG).astype(jnp.int32)
    return out_values, out_lengths, out_offsets


# speedup vs baseline: 30.9974x; 2.7925x over previous
import functools

import jax
import jax.numpy as jnp
from jax import lax
from jax.experimental import pallas as pl
from jax.experimental.pallas import tpu as pltpu
from jax.experimental.pallas import tpu_sc as plsc

B = 16
L = 2048
D = 256

NC = 2
NS = 16
SEG = 2 * L + 1
HALF = L // 2
CHUNK = 64
NCHUNK = HALF // CHUNK


def _sc_interleave(item_values, action_values, contextual_values):
    mesh = plsc.VectorSubcoreMesh(core_axis_name="c", subcore_axis_name="s")
    out_rows = B * SEG
    nbuf = 4
    nt = 2 * NCHUNK

    @functools.partial(
        pl.kernel,
        mesh=mesh,
        out_type=jax.ShapeDtypeStruct((out_rows, D), jnp.float32),
        scratch_types=(
            [pltpu.VMEM((CHUNK, D), jnp.float32) for _ in range(nbuf)]
            + [pltpu.VMEM((CHUNK,), jnp.int32) for _ in range(nbuf)]
            + [pltpu.VMEM((1, D), jnp.float32)]
            + [pltpu.SemaphoreType.DMA for _ in range(2 * nbuf)]
        ),
    )
    def k(item_hbm, action_hbm, ctx_hbm, out_hbm, *scr):
        bufs = scr[0:nbuf]
        idxs = scr[nbuf:2 * nbuf]
        ctx_buf = scr[2 * nbuf]
        gsem = scr[2 * nbuf + 1:2 * nbuf + 1 + nbuf]
        ssem = scr[2 * nbuf + 1 + nbuf:]
        wid = lax.axis_index("s") * NC + lax.axis_index("c")
        b = wid // 2
        h = wid % 2
        src_base = b * L + h * HALF
        out_base = b * SEG + 1 + h * (2 * HALF)
        lane = lax.iota(jnp.int32, 16)

        def start_gather(t):
            ref = item_hbm if t % 2 == 0 else action_hbm
            src = src_base + (t // 2) * CHUNK
            return pltpu.async_copy(ref.at[pl.ds(src, CHUNK)],
                                    bufs[t % nbuf], gsem[t % nbuf])

        gath = {t: start_gather(t) for t in range(min(2, nt))}
        scat = {}
        for t in range(nt):
            if t + 2 < nt:
                if t >= 2:
                    scat[t - 2].wait()
                gath[t + 2] = start_gather(t + 2)
            gath[t].wait()
            s = t % nbuf
            a = t % 2
            base = out_base + a + 2 * (t // 2) * CHUNK
            for i in range(CHUNK // 16):
                idxs[s][pl.ds(i * 16, 16)] = base + 2 * (i * 16 + lane)
            scat[t] = pltpu.async_copy(bufs[s], out_hbm.at[idxs[s]], ssem[s])
        for t in range(max(0, nt - nbuf), nt):
            scat[t].wait()

        @pl.when(wid < B)
        def _():
            pltpu.sync_copy(ctx_hbm.at[pl.ds(wid, 1)], ctx_buf)
            pltpu.sync_copy(ctx_buf, out_hbm.at[pl.ds(wid * SEG, 1)])

    return k(item_values, action_values, contextual_values)


def kernel(item_values, action_values, contextual_values):
    out_values = _sc_interleave(item_values, action_values, contextual_values)
    out_lengths = jnp.full((B,), SEG, dtype=jnp.int32)
    out_offsets = (jnp.arange(B + 1, dtype=jnp.int32) * SEG).astype(jnp.int32)
    return out_values, out_lengths, out_offsets


# speedup vs baseline: 31.3179x; 1.0103x over previous
import functools

import jax
import jax.numpy as jnp
from jax import lax
from jax.experimental import pallas as pl
from jax.experimental.pallas import tpu as pltpu
from jax.experimental.pallas import tpu_sc as plsc

B = 16
L = 2048
D = 256

NC = 2
NS = 16
SEG = 2 * L + 1
HALF = L // 2
CHUNK = 128
NCHUNK = HALF // CHUNK


def _sc_preprocess(item_values, action_values, contextual_values):
    mesh = plsc.VectorSubcoreMesh(core_axis_name="c", subcore_axis_name="s")
    out_rows = B * SEG
    nbuf = 3
    nt = 2 * NCHUNK

    @functools.partial(
        pl.kernel,
        mesh=mesh,
        out_type=(
            jax.ShapeDtypeStruct((out_rows, D), jnp.float32),
            jax.ShapeDtypeStruct((B,), jnp.int32),
            jax.ShapeDtypeStruct((B + 1,), jnp.int32),
        ),
        scratch_types=(
            [pltpu.VMEM((CHUNK, D), jnp.float32) for _ in range(nbuf)]
            + [pltpu.VMEM((CHUNK,), jnp.int32) for _ in range(nbuf)]
            + [pltpu.VMEM((1, D), jnp.float32)]
            + [pltpu.VMEM((16,), jnp.int32), pltpu.VMEM((32,), jnp.int32)]
            + [pltpu.SemaphoreType.DMA for _ in range(2 * nbuf + 1)]
        ),
    )
    def k(item_hbm, action_hbm, ctx_hbm, out_hbm, len_hbm, off_hbm, *scr):
        bufs = scr[0:nbuf]
        idxs = scr[nbuf:2 * nbuf]
        ctx_buf = scr[2 * nbuf]
        len_buf = scr[2 * nbuf + 1]
        off_buf = scr[2 * nbuf + 2]
        gsem = scr[2 * nbuf + 3:2 * nbuf + 3 + nbuf]
        ssem = scr[2 * nbuf + 3 + nbuf:2 * nbuf + 3 + 2 * nbuf]
        msem = scr[2 * nbuf + 3 + 2 * nbuf]
        wid = lax.axis_index("s") * NC + lax.axis_index("c")
        b = wid // 2
        h = wid % 2
        src_base = b * L + h * HALF
        out_base = b * SEG + 1 + h * (2 * HALF)
        lane = lax.iota(jnp.int32, 16)

        def start_gather(t):
            ref = item_hbm if t % 2 == 0 else action_hbm
            src = src_base + (t // 2) * CHUNK
            return pltpu.async_copy(ref.at[pl.ds(src, CHUNK)],
                                    bufs[t % nbuf], gsem[t % nbuf])

        gath = {t: start_gather(t) for t in range(min(nbuf, nt))}
        scat = {}
        for t in range(nt):
            if t >= 1 and t + 2 < nt:
                scat[t - 1].wait()
                gath[t + 2] = start_gather(t + 2)
            gath[t].wait()
            s = t % nbuf
            a = t % 2
            base = out_base + a + 2 * (t // 2) * CHUNK
            for i in range(CHUNK // 16):
                idxs[s][pl.ds(i * 16, 16)] = base + 2 * (i * 16 + lane)
            scat[t] = pltpu.async_copy(bufs[s], out_hbm.at[idxs[s]], ssem[s])
        for t in range(max(0, nt - nbuf), nt):
            scat[t].wait()

        @pl.when(wid < B)
        def _():
            pltpu.sync_copy(ctx_hbm.at[pl.ds(wid, 1)], ctx_buf)
            pltpu.sync_copy(ctx_buf, out_hbm.at[pl.ds(wid * SEG, 1)])

        @pl.when(wid == 0)
        def _():
            len_buf[...] = lane * 0 + SEG
            off_buf[pl.ds(0, 16)] = SEG * lane
            off_buf[pl.ds(16, 16)] = SEG * (16 + lane)
            cp_l = pltpu.async_copy(len_buf, len_hbm, msem)
            cp_l.wait()
            cp_o = pltpu.async_copy(off_buf.at[pl.ds(0, B + 1)], off_hbm,
                                    msem)
            cp_o.wait()

    return k(item_values, action_values, contextual_values)


def kernel(item_values, action_values, contextual_values):
    out_values, out_lengths, out_offsets = _sc_preprocess(
        item_values, action_values, contextual_values)
    return out_values, out_lengths, out_offsets


# speedup vs baseline: 31.3269x; 1.0003x over previous
import functools

import jax
import jax.numpy as jnp
from jax import lax
from jax.experimental import pallas as pl
from jax.experimental.pallas import tpu as pltpu
from jax.experimental.pallas import tpu_sc as plsc

B = 16
L = 2048
D = 256

NC = 2
NS = 16
SEG = 2 * L + 1
HALF = L // 2
CHUNK = 128
NCHUNK = HALF // CHUNK


def _sc_preprocess(item_values, action_values, contextual_values):
    mesh = plsc.VectorSubcoreMesh(core_axis_name="c", subcore_axis_name="s")
    out_rows = B * SEG
    nbuf = 3
    nt = 2 * NCHUNK

    @functools.partial(
        pl.kernel,
        mesh=mesh,
        out_type=(
            jax.ShapeDtypeStruct((out_rows, D), jnp.float32),
            jax.ShapeDtypeStruct((B,), jnp.int32),
            jax.ShapeDtypeStruct((B + 1,), jnp.int32),
        ),
        scratch_types=(
            [pltpu.VMEM((CHUNK, D), jnp.float32) for _ in range(nbuf)]
            + [pltpu.VMEM((CHUNK,), jnp.int32) for _ in range(nbuf)]
            + [pltpu.VMEM((1, D), jnp.float32)]
            + [pltpu.VMEM((16,), jnp.int32), pltpu.VMEM((32,), jnp.int32)]
            + [pltpu.SemaphoreType.DMA for _ in range(2 * nbuf + 3)]
        ),
    )
    def k(item_hbm, action_hbm, ctx_hbm, out_hbm, len_hbm, off_hbm, *scr):
        bufs = scr[0:nbuf]
        idxs = scr[nbuf:2 * nbuf]
        ctx_buf = scr[2 * nbuf]
        len_buf = scr[2 * nbuf + 1]
        off_buf = scr[2 * nbuf + 2]
        gsem = scr[2 * nbuf + 3:2 * nbuf + 3 + nbuf]
        ssem = scr[2 * nbuf + 3 + nbuf:2 * nbuf + 3 + 2 * nbuf]
        csem_g = scr[2 * nbuf + 3 + 2 * nbuf]
        csem_s = scr[2 * nbuf + 3 + 2 * nbuf + 1]
        msem = scr[2 * nbuf + 3 + 2 * nbuf + 2]
        wid = lax.axis_index("s") * NC + lax.axis_index("c")
        b = wid // 2
        h = wid % 2
        src_base = b * L + h * HALF
        out_base = b * SEG + 1 + h * (2 * HALF)
        lane = lax.iota(jnp.int32, 16)

        is_ctx = wid < B

        @pl.when(is_ctx)
        def _():
            pltpu.async_copy(ctx_hbm.at[pl.ds(wid, 1)], ctx_buf,
                             csem_g).wait()
            pltpu.async_copy(ctx_buf, out_hbm.at[pl.ds(wid * SEG, 1)],
                             csem_s)

        @pl.when(wid == 0)
        def _():
            len_buf[...] = lane * 0 + SEG
            off_buf[pl.ds(0, 16)] = SEG * lane
            off_buf[pl.ds(16, 16)] = SEG * (16 + lane)
            pltpu.async_copy(len_buf, len_hbm, msem)
            pltpu.async_copy(off_buf.at[pl.ds(0, B + 1)], off_hbm, msem)

        def start_gather(t):
            ref = item_hbm if t % 2 == 0 else action_hbm
            src = src_base + (t // 2) * CHUNK
            return pltpu.async_copy(ref.at[pl.ds(src, CHUNK)],
                                    bufs[t % nbuf], gsem[t % nbuf])

        gath = {t: start_gather(t) for t in range(min(nbuf, nt))}
        scat = {}
        for t in range(nt):
            if t >= 1 and t + 2 < nt:
                scat[t - 1].wait()
                gath[t + 2] = start_gather(t + 2)
            gath[t].wait()
            s = t % nbuf
            a = t % 2
            base = out_base + a + 2 * (t // 2) * CHUNK
            for i in range(CHUNK // 16):
                idxs[s][pl.ds(i * 16, 16)] = base + 2 * (i * 16 + lane)
            scat[t] = pltpu.async_copy(bufs[s], out_hbm.at[idxs[s]], ssem[s])
        for t in range(max(0, nt - nbuf), nt):
            scat[t].wait()

        @pl.when(is_ctx)
        def _():
            pltpu.make_async_copy(ctx_buf, out_hbm.at[pl.ds(wid * SEG, 1)],
                                  csem_s).wait()

        @pl.when(wid == 0)
        def _():
            pltpu.make_async_copy(len_buf, len_hbm, msem).wait()
            pltpu.make_async_copy(off_buf.at[pl.ds(0, B + 1)], off_hbm,
                                  msem).wait()

    return k(item_values, action_values, contextual_values)


def kernel(item_values, action_values, contextual_values):
    out_values, out_lengths, out_offsets = _sc_preprocess(
        item_values, action_values, contextual_values)
    return out_values, out_lengths, out_offsets


# speedup vs baseline: 31.3920x; 1.0021x over previous
import functools

import jax
import jax.numpy as jnp
from jax import lax
from jax.experimental import pallas as pl
from jax.experimental.pallas import tpu as pltpu
from jax.experimental.pallas import tpu_sc as plsc

B = 16
L = 2048
D = 256

NC = 2
NS = 16
SEG = 2 * L + 1
HALF = L // 2
CHUNK = 128
NCHUNK = HALF // CHUNK


def _sc_preprocess(item_values, action_values, contextual_values):
    mesh = plsc.VectorSubcoreMesh(core_axis_name="c", subcore_axis_name="s")
    out_rows = B * SEG
    nbuf = 3
    nt = 2 * NCHUNK

    @functools.partial(
        pl.kernel,
        mesh=mesh,
        out_type=(
            jax.ShapeDtypeStruct((out_rows, D), jnp.float32),
            jax.ShapeDtypeStruct((B,), jnp.int32),
            jax.ShapeDtypeStruct((B + 1,), jnp.int32),
        ),
        scratch_types=(
            [pltpu.VMEM((CHUNK, D), jnp.float32) for _ in range(nbuf)]
            + [pltpu.VMEM((CHUNK,), jnp.int32) for _ in range(nt)]
            + [pltpu.VMEM((1, D), jnp.float32)]
            + [pltpu.VMEM((16,), jnp.int32), pltpu.VMEM((32,), jnp.int32)]
            + [pltpu.SemaphoreType.DMA for _ in range(2 * nbuf + 3)]
        ),
    )
    def k(item_hbm, action_hbm, ctx_hbm, out_hbm, len_hbm, off_hbm, *scr):
        bufs = scr[0:nbuf]
        idxs = scr[nbuf:nbuf + nt]
        base_i = nbuf + nt
        ctx_buf = scr[base_i]
        len_buf = scr[base_i + 1]
        off_buf = scr[base_i + 2]
        gsem = scr[base_i + 3:base_i + 3 + nbuf]
        ssem = scr[base_i + 3 + nbuf:base_i + 3 + 2 * nbuf]
        csem_g = scr[base_i + 3 + 2 * nbuf]
        csem_s = scr[base_i + 3 + 2 * nbuf + 1]
        msem = scr[base_i + 3 + 2 * nbuf + 2]
        wid = lax.axis_index("s") * NC + lax.axis_index("c")
        b = wid // 2
        h = wid % 2
        src_base = b * L + h * HALF
        out_base = b * SEG + 1 + h * (2 * HALF)
        lane = lax.iota(jnp.int32, 16)

        is_ctx = wid < B

        @pl.when(is_ctx)
        def _():
            pltpu.async_copy(ctx_hbm.at[pl.ds(wid, 1)], ctx_buf,
                             csem_g).wait()
            pltpu.async_copy(ctx_buf, out_hbm.at[pl.ds(wid * SEG, 1)],
                             csem_s)

        @pl.when(wid == 0)
        def _():
            len_buf[...] = lane * 0 + SEG
            off_buf[pl.ds(0, 16)] = SEG * lane
            off_buf[pl.ds(16, 16)] = SEG * (16 + lane)
            pltpu.async_copy(len_buf, len_hbm, msem)
            pltpu.async_copy(off_buf.at[pl.ds(0, B + 1)], off_hbm, msem)

        def start_gather(t):
            ref = item_hbm if t % 2 == 0 else action_hbm
            src = src_base + (t // 2) * CHUNK
            return pltpu.async_copy(ref.at[pl.ds(src, CHUNK)],
                                    bufs[t % nbuf], gsem[t % nbuf])

        gath = {t: start_gather(t) for t in range(min(nbuf, nt))}
        for t in range(nt):
            base = out_base + (t % 2) + 2 * (t // 2) * CHUNK
            for i in range(CHUNK // 16):
                idxs[t][pl.ds(i * 16, 16)] = base + 2 * (i * 16 + lane)
        scat = {}
        for t in range(nt):
            if t >= 1 and t + 2 < nt:
                scat[t - 1].wait()
                gath[t + 2] = start_gather(t + 2)
            gath[t].wait()
            s = t % nbuf
            scat[t] = pltpu.async_copy(bufs[s], out_hbm.at[idxs[t]], ssem[s])
        for t in range(max(0, nt - nbuf), nt):
            scat[t].wait()

        @pl.when(is_ctx)
        def _():
            pltpu.make_async_copy(ctx_buf, out_hbm.at[pl.ds(wid * SEG, 1)],
                                  csem_s).wait()

        @pl.when(wid == 0)
        def _():
            pltpu.make_async_copy(len_buf, len_hbm, msem).wait()
            pltpu.make_async_copy(off_buf.at[pl.ds(0, B + 1)], off_hbm,
                                  msem).wait()

    return k(item_values, action_values, contextual_values)


def kernel(item_values, action_values, contextual_values):
    out_values, out_lengths, out_offsets = _sc_preprocess(
        item_values, action_values, contextual_values)
    return out_values, out_lengths, out_offsets
